# baseline (speedup 1.0000x reference)
import functools

import jax
import jax.numpy as jnp
from jax import lax
from jax.experimental import pallas as pl
from jax.experimental.pallas import tpu as pltpu
from jax.experimental.pallas import tpu_sc as plsc

MAX_LEN = 2048
EMBED_DIM = 32
NUM_WORKERS = 32
CHUNK = 2048
LANES = 16


def _sc_lookup(positions_flat, weight):
    total = positions_flat.shape[0]
    per_worker = total // NUM_WORKERS
    num_chunks = per_worker // CHUNK

    mesh = plsc.VectorSubcoreMesh(core_axis_name="c", subcore_axis_name="s")

    @functools.partial(
        pl.kernel,
        mesh=mesh,
        out_type=jax.ShapeDtypeStruct((total, EMBED_DIM), jnp.float32),
        scratch_types=[
            pltpu.VMEM((CHUNK,), jnp.int32),
            pltpu.VMEM((CHUNK, EMBED_DIM), jnp.float32),
            pltpu.SemaphoreType.DMA,
        ],
        compiler_params=pltpu.CompilerParams(use_tc_tiling_on_sc=False),
    )
    def k(pos_hbm, tab_hbm, out_hbm, idx_v, rows_v, sem):
        wid = lax.axis_index("s") * 2 + lax.axis_index("c")
        base = wid * per_worker

        def chunk_body(ch, carry):
            off = base + ch * CHUNK
            pltpu.sync_copy(pos_hbm.at[pl.ds(off, CHUNK)], idx_v)

            def xform(i, c):
                v = idx_v[pl.ds(i * LANES, LANES)]
                v = jnp.clip(v, -MAX_LEN + 1, MAX_LEN - 1) + (MAX_LEN - 1)
                idx_v[pl.ds(i * LANES, LANES)] = v
                return c

            lax.fori_loop(0, CHUNK // LANES, xform, 0)
            pltpu.async_copy(tab_hbm.at[idx_v], rows_v, sem).wait()
            pltpu.sync_copy(rows_v, out_hbm.at[pl.ds(off, CHUNK)])
            return carry

        lax.fori_loop(0, num_chunks, chunk_body, 0)

    return k(positions_flat, weight)


def kernel(positions, weight):
    n_i, n_j = positions.shape
    flat = positions.reshape(n_i * n_j)
    out = _sc_lookup(flat, weight)
    return out.reshape(n_i, n_j, EMBED_DIM)
